# Initial kernel scaffold; baseline (speedup 1.0000x reference)
import jax
import jax.numpy as jnp
from jax.experimental import pallas as pl


def kernel(x, edge_index, batch, edge_attr, params):
    raise NotImplementedError("write your pallas kernel here")


# speedup vs baseline: 3.2361x; 3.2361x over previous
import functools

import jax
import jax.numpy as jnp
from jax import lax
from jax.experimental import pallas as pl
from jax.experimental.pallas import tpu as pltpu
from jax.experimental.pallas import tpu_sc as plsc

N = 10000
E = 320000
F = 128
NUM_GRAPHS = 8 * 8

NC = 2
NS = 16
NW = NC * NS
EPT = E // NW
CHUNK = 80
NCHUNK = EPT // CHUNK
NPAD = 10240
ROWS_PT = NPAD // NS
ZROWS = 80


def _sc_scatter_kernel(t_hbm, src_hbm, dst_hbm, attr_hbm, out_hbm,
                       acc, srcv, dstv, attrv, rows, zbuf, sem):
    cid = lax.axis_index("c")
    sid = lax.axis_index("s")
    wid = sid * NC + cid

    def zrow(i, carry):
        for j in range(F // 16):
            zbuf[i, pl.ds(j * 16, 16)] = jnp.zeros((16,), jnp.float32)
        return carry
    lax.fori_loop(0, ZROWS, zrow, 0)
    for k in range(ROWS_PT // ZROWS):
        pltpu.sync_copy(zbuf, acc.at[pl.ds(sid * ROWS_PT + k * ZROWS, ZROWS)])
    plsc.subcore_barrier()

    base = wid * EPT

    def chunk_body(i, carry):
        off = base + i * CHUNK
        pltpu.sync_copy(src_hbm.at[pl.ds(off, CHUNK)], srcv)
        pltpu.sync_copy(dst_hbm.at[pl.ds(off, CHUNK)], dstv)
        pltpu.sync_copy(attr_hbm.at[pl.ds(off, CHUNK)], attrv)
        pltpu.async_copy(t_hbm.at[srcv], rows, sem).wait()

        def edge_body(e, c2):
            a = plsc.load_gather(attrv, [jnp.full((16,), e, jnp.int32)])
            for j in range(F // 16):
                rows[e, pl.ds(j * 16, 16)] = rows[e, pl.ds(j * 16, 16)] * a
            return c2
        lax.fori_loop(0, CHUNK, edge_body, 0)

        pltpu.sync_copy(rows, acc.at[dstv], add=True)
        return carry
    lax.fori_loop(0, NCHUNK, chunk_body, 0)

    plsc.subcore_barrier()
    sl = pl.ds(sid * ROWS_PT, ROWS_PT)
    pltpu.sync_copy(acc.at[sl], out_hbm.at[cid, sl])


_sc_scatter = functools.partial(
    pl.kernel,
    mesh=plsc.VectorSubcoreMesh(core_axis_name="c", subcore_axis_name="s"),
    out_type=jax.ShapeDtypeStruct((NC, NPAD, F), jnp.float32),
    scratch_types=[
        pltpu.VMEM_SHARED((NPAD, F), jnp.float32),
        pltpu.VMEM((CHUNK,), jnp.int32),
        pltpu.VMEM((CHUNK,), jnp.int32),
        pltpu.VMEM((CHUNK,), jnp.float32),
        pltpu.VMEM((CHUNK, F), jnp.float32),
        pltpu.VMEM((ZROWS, F), jnp.float32),
        pltpu.SemaphoreType.DMA,
    ],
    compiler_params=pltpu.CompilerParams(needs_layout_passes=False),
)(_sc_scatter_kernel)


def _msg_body(h_ref, w1_ref, b1_ref, w2_ref, b2_ref, t_ref):
    a = jnp.maximum(
        jnp.dot(h_ref[...], w1_ref[...], preferred_element_type=jnp.float32)
        + b1_ref[...], 0.0)
    t_ref[...] = jnp.dot(a, w2_ref[...], preferred_element_type=jnp.float32) \
        + b2_ref[...]


def _msg_tc(h, w1, b1, w2, b2):
    return pl.pallas_call(
        _msg_body,
        out_shape=jax.ShapeDtypeStruct((N, F), jnp.float32),
    )(h, w1, b1, w2, b2)


def _update_body(h_ref, g0_ref, g1_ref, ew_ref, w1h_ref, w1a_ref, b1_ref,
                 w2_ref, b2_ref, gam_ref, bet_ref, out_ref):
    agg = (g0_ref[:N, :] + g1_ref[:N, :]) * ew_ref[...]
    a = jnp.maximum(
        jnp.dot(h_ref[...], w1h_ref[...], preferred_element_type=jnp.float32)
        + jnp.dot(agg, w1a_ref[...], preferred_element_type=jnp.float32)
        + b1_ref[...], 0.0)
    u = jnp.dot(a, w2_ref[...], preferred_element_type=jnp.float32) + b2_ref[...]
    mean = jnp.mean(u, axis=0, keepdims=True)
    var = jnp.mean((u - mean) * (u - mean), axis=0, keepdims=True)
    hn = gam_ref[...] * (u - mean) / jnp.sqrt(var + 1e-5) + bet_ref[...]
    out_ref[...] = jnp.maximum(hn, 0.0)


def _update_tc(h, g0, g1, ew, w1h, w1a, b1, w2, b2, gam, bet):
    return pl.pallas_call(
        _update_body,
        out_shape=jax.ShapeDtypeStruct((N, F), jnp.float32),
    )(h, g0, g1, ew, w1h, w1a, b1, w2, b2, gam, bet)


def _pool_body(h_ref, batch_ref, lw_ref, lb_ref, out_ref):
    gid = lax.broadcasted_iota(jnp.int32, (N, NUM_GRAPHS), 1)
    onehot = jnp.where(batch_ref[...] == gid, 1.0, 0.0).astype(jnp.float32)
    pooled = lax.dot_general(onehot, h_ref[...], (((0,), (0,)), ((), ())),
                             preferred_element_type=jnp.float32)
    counts = lax.dot_general(onehot, jnp.ones((N, 1), jnp.float32),
                             (((0,), (0,)), ((), ())),
                             preferred_element_type=jnp.float32)
    pooled = pooled / jnp.maximum(counts, 1.0)
    out_ref[...] = jnp.dot(pooled, lw_ref[...],
                           preferred_element_type=jnp.float32) + lb_ref[...]


def _pool_tc(h, batch2d, lw, lb):
    return pl.pallas_call(
        _pool_body,
        out_shape=jax.ShapeDtypeStruct((NUM_GRAPHS, lw.shape[1]), jnp.float32),
    )(h, batch2d, lw, lb)


def kernel(x, edge_index, batch, edge_attr, params):
    src = edge_index[0]
    dst = edge_index[1]
    attr = edge_attr.reshape(E)
    batch2d = batch.reshape(N, 1)

    h = x
    for p in params['layers']:
        t = _msg_tc(h, p['msg_W1'], p['msg_b1'].reshape(1, F),
                    p['msg_W2'], p['msg_b2'].reshape(1, F))
        g = _sc_scatter(t, src, dst, attr)
        h = _update_tc(h, g[0], g[1], p['edge_W'],
                       p['upd_W1'][:F], p['upd_W1'][F:],
                       p['upd_b1'].reshape(1, F), p['upd_W2'],
                       p['upd_b2'].reshape(1, F),
                       p['bn_gamma'].reshape(1, F), p['bn_beta'].reshape(1, F))
    return _pool_tc(h, batch2d, params['lin_W'], params['lin_b'].reshape(1, -1))


# speedup vs baseline: 4.4608x; 1.3785x over previous
import functools

import jax
import jax.numpy as jnp
from jax import lax
from jax.experimental import pallas as pl
from jax.experimental.pallas import tpu as pltpu
from jax.experimental.pallas import tpu_sc as plsc

N = 10000
E = 320000
F = 128
NUM_GRAPHS = 8 * 8

NC = 2
NS = 16
NW = NC * NS
EPT = E // NW
CHUNK = 80
NCHUNK = EPT // CHUNK
NPAD = 10240
ROWS_PT = NPAD // NS
ZROWS = 80


def _sc_scatter_kernel(t_hbm, src_hbm, dst_hbm, attr_hbm, out_hbm,
                       acc, srcv, dstv, attrv, rows, zbuf, gsem):
    cid = lax.axis_index("c")
    sid = lax.axis_index("s")
    wid = sid * NC + cid

    def zrow(i, carry):
        for j in range(F // 16):
            zbuf[i, pl.ds(j * 16, 16)] = jnp.zeros((16,), jnp.float32)
        return carry
    lax.fori_loop(0, ZROWS, zrow, 0)
    for k in range(ROWS_PT // ZROWS):
        pltpu.sync_copy(zbuf, acc.at[pl.ds(sid * ROWS_PT + k * ZROWS, ZROWS)])
    plsc.subcore_barrier()

    base = wid * EPT

    def load_idx(i, b):
        off = base + i * CHUNK
        pltpu.sync_copy(src_hbm.at[pl.ds(off, CHUNK)], srcv.at[b])
        pltpu.sync_copy(dst_hbm.at[pl.ds(off, CHUNK)], dstv.at[b])
        pltpu.sync_copy(attr_hbm.at[pl.ds(off, CHUNK)], attrv.at[b])

    def start_gather(b):
        pltpu.async_copy(t_hbm.at[srcv.at[b]], rows.at[b], gsem)

    def wait_gather(b):
        pltpu.make_async_copy(t_hbm.at[srcv.at[b]], rows.at[b], gsem).wait()

    def scale_and_scatter(b):
        def edge_body(q, c2):
            for k in range(4):
                e = q * 4 + k
                a = plsc.load_gather(attrv.at[b], [jnp.full((16,), e, jnp.int32)])
                for j in range(F // 16):
                    rows[b, e, pl.ds(j * 16, 16)] = (
                        rows[b, e, pl.ds(j * 16, 16)] * a)
            return c2
        lax.fori_loop(0, CHUNK // 4, edge_body, 0)
        pltpu.sync_copy(rows.at[b], acc.at[dstv.at[b]], add=True)

    load_idx(0, 0)
    start_gather(0)

    def pipe_body(i, carry):
        for b in range(2):
            wait_gather(b)
            load_idx(2 * i + b + 1, 1 - b)
            start_gather(1 - b)
            scale_and_scatter(b)
        return carry
    lax.fori_loop(0, (NCHUNK - 1) // 2, pipe_body, 0)
    wait_gather(0)
    scale_and_scatter(0)

    plsc.subcore_barrier()
    sl = pl.ds(sid * ROWS_PT, ROWS_PT)
    pltpu.sync_copy(acc.at[sl], out_hbm.at[cid, sl])


_sc_scatter = functools.partial(
    pl.kernel,
    mesh=plsc.VectorSubcoreMesh(core_axis_name="c", subcore_axis_name="s"),
    out_type=jax.ShapeDtypeStruct((NC, NPAD, F), jnp.float32),
    scratch_types=[
        pltpu.VMEM_SHARED((NPAD, F), jnp.float32),
        pltpu.VMEM((2, CHUNK), jnp.int32),
        pltpu.VMEM((2, CHUNK), jnp.int32),
        pltpu.VMEM((2, CHUNK), jnp.float32),
        pltpu.VMEM((2, CHUNK, F), jnp.float32),
        pltpu.VMEM((ZROWS, F), jnp.float32),
        pltpu.SemaphoreType.DMA,
    ],
    compiler_params=pltpu.CompilerParams(needs_layout_passes=False),
)(_sc_scatter_kernel)


def _msg_body(h_ref, w1_ref, b1_ref, w2_ref, b2_ref, t_ref):
    a = jnp.maximum(
        jnp.dot(h_ref[...], w1_ref[...], preferred_element_type=jnp.float32)
        + b1_ref[...], 0.0)
    t_ref[...] = jnp.dot(a, w2_ref[...], preferred_element_type=jnp.float32) \
        + b2_ref[...]


def _msg_tc(h, w1, b1, w2, b2):
    return pl.pallas_call(
        _msg_body,
        out_shape=jax.ShapeDtypeStruct((N, F), jnp.float32),
    )(h, w1, b1, w2, b2)


def _update_body(h_ref, g0_ref, g1_ref, ew_ref, w1h_ref, w1a_ref, b1_ref,
                 w2_ref, b2_ref, gam_ref, bet_ref, out_ref):
    agg = (g0_ref[:N, :] + g1_ref[:N, :]) * ew_ref[...]
    a = jnp.maximum(
        jnp.dot(h_ref[...], w1h_ref[...], preferred_element_type=jnp.float32)
        + jnp.dot(agg, w1a_ref[...], preferred_element_type=jnp.float32)
        + b1_ref[...], 0.0)
    u = jnp.dot(a, w2_ref[...], preferred_element_type=jnp.float32) + b2_ref[...]
    mean = jnp.mean(u, axis=0, keepdims=True)
    var = jnp.mean((u - mean) * (u - mean), axis=0, keepdims=True)
    hn = gam_ref[...] * (u - mean) / jnp.sqrt(var + 1e-5) + bet_ref[...]
    out_ref[...] = jnp.maximum(hn, 0.0)


def _update_tc(h, g0, g1, ew, w1h, w1a, b1, w2, b2, gam, bet):
    return pl.pallas_call(
        _update_body,
        out_shape=jax.ShapeDtypeStruct((N, F), jnp.float32),
    )(h, g0, g1, ew, w1h, w1a, b1, w2, b2, gam, bet)


def _pool_body(h_ref, batch_ref, lw_ref, lb_ref, out_ref):
    gid = lax.broadcasted_iota(jnp.int32, (N, NUM_GRAPHS), 1)
    onehot = jnp.where(batch_ref[...] == gid, 1.0, 0.0).astype(jnp.float32)
    pooled = lax.dot_general(onehot, h_ref[...], (((0,), (0,)), ((), ())),
                             preferred_element_type=jnp.float32)
    counts = lax.dot_general(onehot, jnp.ones((N, 1), jnp.float32),
                             (((0,), (0,)), ((), ())),
                             preferred_element_type=jnp.float32)
    pooled = pooled / jnp.maximum(counts, 1.0)
    out_ref[...] = jnp.dot(pooled, lw_ref[...],
                           preferred_element_type=jnp.float32) + lb_ref[...]


def _pool_tc(h, batch2d, lw, lb):
    return pl.pallas_call(
        _pool_body,
        out_shape=jax.ShapeDtypeStruct((NUM_GRAPHS, lw.shape[1]), jnp.float32),
    )(h, batch2d, lw, lb)


def kernel(x, edge_index, batch, edge_attr, params):
    src = edge_index[0]
    dst = edge_index[1]
    attr = edge_attr.reshape(E)
    batch2d = batch.reshape(N, 1)

    h = x
    for p in params['layers']:
        t = _msg_tc(h, p['msg_W1'], p['msg_b1'].reshape(1, F),
                    p['msg_W2'], p['msg_b2'].reshape(1, F))
        g = _sc_scatter(t, src, dst, attr)
        h = _update_tc(h, g[0], g[1], p['edge_W'],
                       p['upd_W1'][:F], p['upd_W1'][F:],
                       p['upd_b1'].reshape(1, F), p['upd_W2'],
                       p['upd_b2'].reshape(1, F),
                       p['bn_gamma'].reshape(1, F), p['bn_beta'].reshape(1, F))
    return _pool_tc(h, batch2d, params['lin_W'], params['lin_b'].reshape(1, -1))


# speedup vs baseline: 4.6316x; 1.0383x over previous
import functools

import jax
import jax.numpy as jnp
from jax import lax
from jax.experimental import pallas as pl
from jax.experimental.pallas import tpu as pltpu
from jax.experimental.pallas import tpu_sc as plsc

N = 10000
E = 320000
F = 128
NUM_GRAPHS = 8 * 8

NC = 2
NS = 16
NW = NC * NS
EPT = E // NW
CHUNK = 80
NCHUNK = EPT // CHUNK
NPAD = 10240
ROWS_PT = NPAD // NS
ZROWS = 80


def _sc_scatter_kernel(t_hbm, src_hbm, dst_hbm, attr_hbm, out_hbm,
                       acc, srcv, dstv, attrv, rows, zbuf, gsem):
    cid = lax.axis_index("c")
    sid = lax.axis_index("s")
    wid = sid * NC + cid

    def zrow(i, carry):
        for j in range(F // 16):
            zbuf[i, pl.ds(j * 16, 16)] = jnp.zeros((16,), jnp.float32)
        return carry
    lax.fori_loop(0, ZROWS, zrow, 0)
    for k in range(ROWS_PT // ZROWS):
        pltpu.sync_copy(zbuf, acc.at[pl.ds(sid * ROWS_PT + k * ZROWS, ZROWS)])
    plsc.subcore_barrier()

    base = wid * EPT

    def load_idx(i, b):
        off = base + i * CHUNK
        pltpu.sync_copy(src_hbm.at[pl.ds(off, CHUNK)], srcv.at[b])
        pltpu.sync_copy(dst_hbm.at[pl.ds(off, CHUNK)], dstv.at[b])
        pltpu.sync_copy(attr_hbm.at[pl.ds(off, CHUNK)], attrv.at[b])

    def start_gather(b):
        pltpu.async_copy(t_hbm.at[srcv.at[b]], rows.at[b], gsem)

    def wait_gather(b):
        pltpu.make_async_copy(t_hbm.at[srcv.at[b]], rows.at[b], gsem).wait()

    def scale_and_scatter(b):
        @plsc.parallel_loop(0, CHUNK, step=1, unroll=8)
        def _scale(e):
            a = plsc.load_gather(attrv.at[b], [jnp.full((16,), e, jnp.int32)])
            for j in range(F // 16):
                rows[b, e, pl.ds(j * 16, 16)] = rows[b, e, pl.ds(j * 16, 16)] * a
        pltpu.sync_copy(rows.at[b], acc.at[dstv.at[b]], add=True)

    load_idx(0, 0)
    start_gather(0)

    def pipe_body(i, carry):
        for b in range(2):
            wait_gather(b)
            load_idx(2 * i + b + 1, 1 - b)
            start_gather(1 - b)
            scale_and_scatter(b)
        return carry
    lax.fori_loop(0, (NCHUNK - 1) // 2, pipe_body, 0)
    wait_gather(0)
    scale_and_scatter(0)

    plsc.subcore_barrier()
    sl = pl.ds(sid * ROWS_PT, ROWS_PT)
    pltpu.sync_copy(acc.at[sl], out_hbm.at[cid, sl])


_sc_scatter = functools.partial(
    pl.kernel,
    mesh=plsc.VectorSubcoreMesh(core_axis_name="c", subcore_axis_name="s"),
    out_type=jax.ShapeDtypeStruct((NC, NPAD, F), jnp.float32),
    scratch_types=[
        pltpu.VMEM_SHARED((NPAD, F), jnp.float32),
        pltpu.VMEM((2, CHUNK), jnp.int32),
        pltpu.VMEM((2, CHUNK), jnp.int32),
        pltpu.VMEM((2, CHUNK), jnp.float32),
        pltpu.VMEM((2, CHUNK, F), jnp.float32),
        pltpu.VMEM((ZROWS, F), jnp.float32),
        pltpu.SemaphoreType.DMA,
    ],
    compiler_params=pltpu.CompilerParams(needs_layout_passes=False),
)(_sc_scatter_kernel)


def _msg_body(h_ref, w1_ref, b1_ref, w2_ref, b2_ref, t_ref):
    a = jnp.maximum(
        jnp.dot(h_ref[...], w1_ref[...], preferred_element_type=jnp.float32)
        + b1_ref[...], 0.0)
    t_ref[...] = jnp.dot(a, w2_ref[...], preferred_element_type=jnp.float32) \
        + b2_ref[...]


def _msg_tc(h, w1, b1, w2, b2):
    return pl.pallas_call(
        _msg_body,
        out_shape=jax.ShapeDtypeStruct((N, F), jnp.float32),
    )(h, w1, b1, w2, b2)


def _update_body(h_ref, g0_ref, g1_ref, ew_ref, w1h_ref, w1a_ref, b1_ref,
                 w2_ref, b2_ref, gam_ref, bet_ref, out_ref):
    agg = (g0_ref[:N, :] + g1_ref[:N, :]) * ew_ref[...]
    a = jnp.maximum(
        jnp.dot(h_ref[...], w1h_ref[...], preferred_element_type=jnp.float32)
        + jnp.dot(agg, w1a_ref[...], preferred_element_type=jnp.float32)
        + b1_ref[...], 0.0)
    u = jnp.dot(a, w2_ref[...], preferred_element_type=jnp.float32) + b2_ref[...]
    mean = jnp.mean(u, axis=0, keepdims=True)
    var = jnp.mean((u - mean) * (u - mean), axis=0, keepdims=True)
    hn = gam_ref[...] * (u - mean) / jnp.sqrt(var + 1e-5) + bet_ref[...]
    out_ref[...] = jnp.maximum(hn, 0.0)


def _update_tc(h, g0, g1, ew, w1h, w1a, b1, w2, b2, gam, bet):
    return pl.pallas_call(
        _update_body,
        out_shape=jax.ShapeDtypeStruct((N, F), jnp.float32),
    )(h, g0, g1, ew, w1h, w1a, b1, w2, b2, gam, bet)


def _pool_body(h_ref, batch_ref, lw_ref, lb_ref, out_ref):
    gid = lax.broadcasted_iota(jnp.int32, (N, NUM_GRAPHS), 1)
    onehot = jnp.where(batch_ref[...] == gid, 1.0, 0.0).astype(jnp.float32)
    pooled = lax.dot_general(onehot, h_ref[...], (((0,), (0,)), ((), ())),
                             preferred_element_type=jnp.float32)
    counts = lax.dot_general(onehot, jnp.ones((N, 1), jnp.float32),
                             (((0,), (0,)), ((), ())),
                             preferred_element_type=jnp.float32)
    pooled = pooled / jnp.maximum(counts, 1.0)
    out_ref[...] = jnp.dot(pooled, lw_ref[...],
                           preferred_element_type=jnp.float32) + lb_ref[...]


def _pool_tc(h, batch2d, lw, lb):
    return pl.pallas_call(
        _pool_body,
        out_shape=jax.ShapeDtypeStruct((NUM_GRAPHS, lw.shape[1]), jnp.float32),
    )(h, batch2d, lw, lb)


def kernel(x, edge_index, batch, edge_attr, params):
    src = edge_index[0]
    dst = edge_index[1]
    attr = edge_attr.reshape(E)
    batch2d = batch.reshape(N, 1)

    h = x
    for p in params['layers']:
        t = _msg_tc(h, p['msg_W1'], p['msg_b1'].reshape(1, F),
                    p['msg_W2'], p['msg_b2'].reshape(1, F))
        g = _sc_scatter(t, src, dst, attr)
        h = _update_tc(h, g[0], g[1], p['edge_W'],
                       p['upd_W1'][:F], p['upd_W1'][F:],
                       p['upd_b1'].reshape(1, F), p['upd_W2'],
                       p['upd_b2'].reshape(1, F),
                       p['bn_gamma'].reshape(1, F), p['bn_beta'].reshape(1, F))
    return _pool_tc(h, batch2d, params['lin_W'], params['lin_b'].reshape(1, -1))


# speedup vs baseline: 6.8355x; 1.4758x over previous
import functools

import jax
import jax.numpy as jnp
from jax import lax
from jax.experimental import pallas as pl
from jax.experimental.pallas import tpu as pltpu
from jax.experimental.pallas import tpu_sc as plsc

N = 10000
E = 320000
F = 128
NUM_GRAPHS = 8 * 8

NC = 2
NS = 16
NW = NC * NS
EPT = E // NW
CHUNK = 80
NCHUNK = EPT // CHUNK
NPAD = 10240
ROWS_PT = NPAD // NS
ZROWS = 80


def _sc_scatter_kernel(t_hbm, idx_hbm, out_hbm, acc, idxc, rows, gsem, isem):
    cid = lax.axis_index("c")
    sid = lax.axis_index("s")
    wid = sid * NC + cid

    @plsc.parallel_loop(0, CHUNK, step=1, unroll=8)
    def _zrow(i):
        for j in range(F // 16):
            rows[0, i, pl.ds(j * 16, 16)] = jnp.zeros((16,), jnp.float32)
    for k in range(ROWS_PT // ZROWS):
        pltpu.sync_copy(rows.at[0], acc.at[pl.ds(sid * ROWS_PT + k * ZROWS, ZROWS)])
    plsc.subcore_barrier()

    def start_idx(i, b):
        pltpu.async_copy(idx_hbm.at[wid, i], idxc.at[b], isem)

    def wait_idx(i, b):
        pltpu.make_async_copy(idx_hbm.at[wid, i], idxc.at[b], isem).wait()

    def start_gather(i, b):
        pltpu.async_copy(t_hbm.at[idxc.at[b, 0]], rows.at[b], gsem)

    def wait_gather(i, b):
        pltpu.make_async_copy(t_hbm.at[idxc.at[b, 0]], rows.at[b], gsem).wait()

    def scale_and_scatter(i, b):
        @plsc.parallel_loop(0, CHUNK, step=1, unroll=8)
        def _scale(e):
            a = plsc.bitcast(
                plsc.load_gather(idxc.at[b, 2], [jnp.full((16,), e, jnp.int32)]),
                jnp.float32)
            for j in range(F // 16):
                rows[b, e, pl.ds(j * 16, 16)] = rows[b, e, pl.ds(j * 16, 16)] * a
        pltpu.sync_copy(rows.at[b], acc.at[idxc.at[b, 1]], add=True)

    pltpu.sync_copy(idx_hbm.at[wid, 0], idxc.at[0])
    start_idx(1, 1)
    start_gather(0, 0)

    def pipe_body(i, carry):
        for b in range(2):
            ch = 2 * i + b
            wait_gather(ch, b)
            wait_idx(ch + 1, 1 - b)
            start_gather(ch + 1, 1 - b)
            scale_and_scatter(ch, b)
            start_idx(jnp.minimum(ch + 2, NCHUNK - 1), b)
        return carry
    lax.fori_loop(0, (NCHUNK - 1) // 2, pipe_body, 0)
    wait_gather(NCHUNK - 1, 0)
    wait_idx(NCHUNK - 1, 1)
    scale_and_scatter(NCHUNK - 1, 0)

    plsc.subcore_barrier()
    sl = pl.ds(sid * ROWS_PT, ROWS_PT)
    pltpu.sync_copy(acc.at[sl], out_hbm.at[cid, sl])


_sc_scatter = functools.partial(
    pl.kernel,
    mesh=plsc.VectorSubcoreMesh(core_axis_name="c", subcore_axis_name="s"),
    out_type=jax.ShapeDtypeStruct((NC, NPAD, F), jnp.float32),
    scratch_types=[
        pltpu.VMEM_SHARED((NPAD, F), jnp.float32),
        pltpu.VMEM((2, 3, CHUNK), jnp.int32),
        pltpu.VMEM((2, CHUNK, F), jnp.float32),
        pltpu.SemaphoreType.DMA,
        pltpu.SemaphoreType.DMA,
    ],
    compiler_params=pltpu.CompilerParams(needs_layout_passes=False),
)(_sc_scatter_kernel)


def _msg_body(h_ref, w1_ref, b1_ref, w2_ref, b2_ref, t_ref):
    a = jnp.maximum(
        jnp.dot(h_ref[...], w1_ref[...], preferred_element_type=jnp.float32)
        + b1_ref[...], 0.0)
    t_ref[...] = jnp.dot(a, w2_ref[...], preferred_element_type=jnp.float32) \
        + b2_ref[...]


def _msg_tc(h, w1, b1, w2, b2):
    return pl.pallas_call(
        _msg_body,
        out_shape=jax.ShapeDtypeStruct((N, F), jnp.float32),
    )(h, w1, b1, w2, b2)


def _update_body(h_ref, g0_ref, g1_ref, ew_ref, w1h_ref, w1a_ref, b1_ref,
                 w2_ref, b2_ref, gam_ref, bet_ref, out_ref):
    agg = (g0_ref[:N, :] + g1_ref[:N, :]) * ew_ref[...]
    a = jnp.maximum(
        jnp.dot(h_ref[...], w1h_ref[...], preferred_element_type=jnp.float32)
        + jnp.dot(agg, w1a_ref[...], preferred_element_type=jnp.float32)
        + b1_ref[...], 0.0)
    u = jnp.dot(a, w2_ref[...], preferred_element_type=jnp.float32) + b2_ref[...]
    mean = jnp.mean(u, axis=0, keepdims=True)
    var = jnp.mean((u - mean) * (u - mean), axis=0, keepdims=True)
    hn = gam_ref[...] * (u - mean) / jnp.sqrt(var + 1e-5) + bet_ref[...]
    out_ref[...] = jnp.maximum(hn, 0.0)


def _update_tc(h, g0, g1, ew, w1h, w1a, b1, w2, b2, gam, bet):
    return pl.pallas_call(
        _update_body,
        out_shape=jax.ShapeDtypeStruct((N, F), jnp.float32),
    )(h, g0, g1, ew, w1h, w1a, b1, w2, b2, gam, bet)


def _pool_body(h_ref, batch_ref, lw_ref, lb_ref, out_ref):
    gid = lax.broadcasted_iota(jnp.int32, (N, NUM_GRAPHS), 1)
    onehot = jnp.where(batch_ref[...] == gid, 1.0, 0.0).astype(jnp.float32)
    pooled = lax.dot_general(onehot, h_ref[...], (((0,), (0,)), ((), ())),
                             preferred_element_type=jnp.float32)
    counts = lax.dot_general(onehot, jnp.ones((N, 1), jnp.float32),
                             (((0,), (0,)), ((), ())),
                             preferred_element_type=jnp.float32)
    pooled = pooled / jnp.maximum(counts, 1.0)
    out_ref[...] = jnp.dot(pooled, lw_ref[...],
                           preferred_element_type=jnp.float32) + lb_ref[...]


def _pool_tc(h, batch2d, lw, lb):
    return pl.pallas_call(
        _pool_body,
        out_shape=jax.ShapeDtypeStruct((NUM_GRAPHS, lw.shape[1]), jnp.float32),
    )(h, batch2d, lw, lb)


def kernel(x, edge_index, batch, edge_attr, params):
    src = edge_index[0].reshape(NW, NCHUNK, CHUNK)
    dst = edge_index[1].reshape(NW, NCHUNK, CHUNK)
    attr_bits = jax.lax.bitcast_convert_type(
        edge_attr.reshape(NW, NCHUNK, CHUNK), jnp.int32)
    idx_packed = jnp.stack([src, dst, attr_bits], axis=2)
    batch2d = batch.reshape(N, 1)

    h = x
    for p in params['layers']:
        t = _msg_tc(h, p['msg_W1'], p['msg_b1'].reshape(1, F),
                    p['msg_W2'], p['msg_b2'].reshape(1, F))
        g = _sc_scatter(t, idx_packed)
        h = _update_tc(h, g[0], g[1], p['edge_W'],
                       p['upd_W1'][:F], p['upd_W1'][F:],
                       p['upd_b1'].reshape(1, F), p['upd_W2'],
                       p['upd_b2'].reshape(1, F),
                       p['bn_gamma'].reshape(1, F), p['bn_beta'].reshape(1, F))
    return _pool_tc(h, batch2d, params['lin_W'], params['lin_b'].reshape(1, -1))


# speedup vs baseline: 6.9542x; 1.0174x over previous
import functools

import jax
import jax.numpy as jnp
from jax import lax
from jax.experimental import pallas as pl
from jax.experimental.pallas import tpu as pltpu
from jax.experimental.pallas import tpu_sc as plsc

N = 10000
E = 320000
F = 128
NUM_GRAPHS = 8 * 8

NC = 2
NS = 16
NW = NC * NS
EPT = E // NW
CHUNK = 80
NCHUNK = EPT // CHUNK
NPAD = 10240
ROWS_PT = NPAD // NS
ZROWS = 80


def _sc_scatter_kernel(t_hbm, idx_hbm, out_hbm, acc, idxc, rows, gsem, isem):
    cid = lax.axis_index("c")
    sid = lax.axis_index("s")
    wid = sid * NC + cid

    @plsc.parallel_loop(0, CHUNK, step=1, unroll=8)
    def _zrow(i):
        for j in range(F // 16):
            rows[0, i, pl.ds(j * 16, 16)] = jnp.zeros((16,), jnp.float32)
    for k in range(ROWS_PT // ZROWS):
        pltpu.sync_copy(rows.at[0], acc.at[pl.ds(sid * ROWS_PT + k * ZROWS, ZROWS)])
    plsc.subcore_barrier()

    def start_idx(i, b):
        pltpu.async_copy(idx_hbm.at[wid, i], idxc.at[b], isem)

    def wait_idx(i, b):
        pltpu.make_async_copy(idx_hbm.at[wid, i], idxc.at[b], isem).wait()

    def start_gather(i, b):
        pltpu.async_copy(t_hbm.at[idxc.at[b, 0]], rows.at[b], gsem)

    def wait_gather(i, b):
        pltpu.make_async_copy(t_hbm.at[idxc.at[b, 0]], rows.at[b], gsem).wait()

    def scale_and_scatter(i, b):
        @plsc.parallel_loop(0, CHUNK, step=1, unroll=8)
        def _scale(e):
            a = plsc.bitcast(
                plsc.load_gather(idxc.at[b, 2], [jnp.full((16,), e, jnp.int32)]),
                jnp.float32)
            for j in range(F // 16):
                rows[b, e, pl.ds(j * 16, 16)] = rows[b, e, pl.ds(j * 16, 16)] * a
        pltpu.sync_copy(rows.at[b], acc.at[idxc.at[b, 1]], add=True)

    pltpu.sync_copy(idx_hbm.at[wid, 0], idxc.at[0])
    start_idx(1, 1)
    start_gather(0, 0)

    def pipe_body(i, carry):
        for b in range(2):
            ch = 2 * i + b
            wait_gather(ch, b)
            wait_idx(ch + 1, 1 - b)
            start_gather(ch + 1, 1 - b)
            scale_and_scatter(ch, b)
            start_idx(jnp.minimum(ch + 2, NCHUNK - 1), b)
        return carry
    lax.fori_loop(0, (NCHUNK - 1) // 2, pipe_body, 0)
    wait_gather(NCHUNK - 1, 0)
    wait_idx(NCHUNK - 1, 1)
    scale_and_scatter(NCHUNK - 1, 0)

    plsc.subcore_barrier()
    sl = pl.ds(sid * ROWS_PT, ROWS_PT)
    pltpu.sync_copy(acc.at[sl], out_hbm.at[cid, sl])


_sc_scatter = functools.partial(
    pl.kernel,
    mesh=plsc.VectorSubcoreMesh(core_axis_name="c", subcore_axis_name="s"),
    out_type=jax.ShapeDtypeStruct((NC, NPAD, F), jnp.float32),
    scratch_types=[
        pltpu.VMEM_SHARED((NPAD, F), jnp.float32),
        pltpu.VMEM((2, 3, CHUNK), jnp.int32),
        pltpu.VMEM((2, CHUNK, F), jnp.float32),
        pltpu.SemaphoreType.DMA,
        pltpu.SemaphoreType.DMA,
    ],
    compiler_params=pltpu.CompilerParams(needs_layout_passes=False),
)(_sc_scatter_kernel)


def _msg_body(h_ref, w1_ref, b1_ref, w2_ref, b2_ref, t_ref):
    a = jnp.maximum(
        jnp.dot(h_ref[...], w1_ref[...], preferred_element_type=jnp.float32)
        + b1_ref[...], 0.0)
    t_ref[...] = jnp.dot(a, w2_ref[...], preferred_element_type=jnp.float32) \
        + b2_ref[...]


def _msg_tc(h, w1, b1, w2, b2):
    return pl.pallas_call(
        _msg_body,
        out_shape=jax.ShapeDtypeStruct((N, F), jnp.float32),
    )(h, w1, b1, w2, b2)


def _update_h(h_ref, g0_ref, g1_ref, ew_ref, w1h_ref, w1a_ref, b1_ref,
              w2_ref, b2_ref, gam_ref, bet_ref):
    agg = (g0_ref[:N, :] + g1_ref[:N, :]) * ew_ref[...]
    a = jnp.maximum(
        jnp.dot(h_ref[...], w1h_ref[...], preferred_element_type=jnp.float32)
        + jnp.dot(agg, w1a_ref[...], preferred_element_type=jnp.float32)
        + b1_ref[...], 0.0)
    u = jnp.dot(a, w2_ref[...], preferred_element_type=jnp.float32) + b2_ref[...]
    mean = jnp.mean(u, axis=0, keepdims=True)
    var = jnp.mean((u - mean) * (u - mean), axis=0, keepdims=True)
    hn = gam_ref[...] * (u - mean) / jnp.sqrt(var + 1e-5) + bet_ref[...]
    return jnp.maximum(hn, 0.0)


def _update_msg_body(h_ref, g0_ref, g1_ref, ew_ref, w1h_ref, w1a_ref, b1_ref,
                     w2_ref, b2_ref, gam_ref, bet_ref,
                     mw1_ref, mb1_ref, mw2_ref, mb2_ref, h_out, t_out):
    hn = _update_h(h_ref, g0_ref, g1_ref, ew_ref, w1h_ref, w1a_ref, b1_ref,
                   w2_ref, b2_ref, gam_ref, bet_ref)
    h_out[...] = hn
    a = jnp.maximum(
        jnp.dot(hn, mw1_ref[...], preferred_element_type=jnp.float32)
        + mb1_ref[...], 0.0)
    t_out[...] = jnp.dot(a, mw2_ref[...], preferred_element_type=jnp.float32) \
        + mb2_ref[...]


def _update_msg_tc(h, g0, g1, ew, w1h, w1a, b1, w2, b2, gam, bet,
                   mw1, mb1, mw2, mb2):
    return pl.pallas_call(
        _update_msg_body,
        out_shape=(jax.ShapeDtypeStruct((N, F), jnp.float32),
                   jax.ShapeDtypeStruct((N, F), jnp.float32)),
    )(h, g0, g1, ew, w1h, w1a, b1, w2, b2, gam, bet, mw1, mb1, mw2, mb2)


def _update_pool_body(h_ref, g0_ref, g1_ref, ew_ref, w1h_ref, w1a_ref, b1_ref,
                      w2_ref, b2_ref, gam_ref, bet_ref,
                      batch_ref, lw_ref, lb_ref, out_ref):
    hn = _update_h(h_ref, g0_ref, g1_ref, ew_ref, w1h_ref, w1a_ref, b1_ref,
                   w2_ref, b2_ref, gam_ref, bet_ref)
    gid = lax.broadcasted_iota(jnp.int32, (N, NUM_GRAPHS), 1)
    onehot = jnp.where(batch_ref[...] == gid, 1.0, 0.0).astype(jnp.float32)
    pooled = lax.dot_general(onehot, hn, (((0,), (0,)), ((), ())),
                             preferred_element_type=jnp.float32)
    counts = lax.dot_general(onehot, jnp.ones((N, 1), jnp.float32),
                             (((0,), (0,)), ((), ())),
                             preferred_element_type=jnp.float32)
    pooled = pooled / jnp.maximum(counts, 1.0)
    out_ref[...] = jnp.dot(pooled, lw_ref[...],
                           preferred_element_type=jnp.float32) + lb_ref[...]


def _update_pool_tc(h, g0, g1, ew, w1h, w1a, b1, w2, b2, gam, bet,
                    batch2d, lw, lb):
    return pl.pallas_call(
        _update_pool_body,
        out_shape=jax.ShapeDtypeStruct((NUM_GRAPHS, lw.shape[1]), jnp.float32),
    )(h, g0, g1, ew, w1h, w1a, b1, w2, b2, gam, bet, batch2d, lw, lb)


def kernel(x, edge_index, batch, edge_attr, params):
    src = edge_index[0].reshape(NW, NCHUNK, CHUNK)
    dst = edge_index[1].reshape(NW, NCHUNK, CHUNK)
    attr_bits = jax.lax.bitcast_convert_type(
        edge_attr.reshape(NW, NCHUNK, CHUNK), jnp.int32)
    idx_packed = jnp.stack([src, dst, attr_bits], axis=2)
    batch2d = batch.reshape(N, 1)

    def upd_args(h, g, p):
        return (h, g[0], g[1], p['edge_W'],
                p['upd_W1'][:F], p['upd_W1'][F:],
                p['upd_b1'].reshape(1, F), p['upd_W2'],
                p['upd_b2'].reshape(1, F),
                p['bn_gamma'].reshape(1, F), p['bn_beta'].reshape(1, F))

    p0, p1, p2 = params['layers']
    h = x
    t = _msg_tc(h, p0['msg_W1'], p0['msg_b1'].reshape(1, F),
                p0['msg_W2'], p0['msg_b2'].reshape(1, F))
    for p, pn in ((p0, p1), (p1, p2)):
        g = _sc_scatter(t, idx_packed)
        h, t = _update_msg_tc(*upd_args(h, g, p),
                              pn['msg_W1'], pn['msg_b1'].reshape(1, F),
                              pn['msg_W2'], pn['msg_b2'].reshape(1, F))
    g = _sc_scatter(t, idx_packed)
    return _update_pool_tc(*upd_args(h, g, p2), batch2d,
                           params['lin_W'], params['lin_b'].reshape(1, -1))
